# Initial kernel scaffold; baseline (speedup 1.0000x reference)
import functools

import jax
import jax.numpy as jnp
from jax import lax
from jax.experimental import pallas as pl
from jax.experimental.pallas import tpu as pltpu
from jax.experimental.pallas import tpu_sc as plsc

B, S = 4096, 200
N = B * S
DW, DC = 128, 64
DO = DW + DC
NW = 32
TOK_PER_W = N // NW
CH = 128
NCHUNK = TOK_PER_W // CH


def _encode(iw_hbm, ic_hbm, wt_hbm, ct_hbm, out_hbm,
            iwv, icv, wbuf, cbuf, sem):
    wid = lax.axis_index("s") * 2 + lax.axis_index("c")
    w_base = wid * TOK_PER_W

    @pl.loop(0, NCHUNK)
    def _chunk(j):
        base = w_base + j * CH
        pltpu.sync_copy(iw_hbm.at[pl.ds(base, CH)], iwv)
        pltpu.sync_copy(ic_hbm.at[pl.ds(base, CH)], icv)
        cp_w = pltpu.async_copy(wt_hbm.at[iwv], wbuf, sem)
        cp_c = pltpu.async_copy(ct_hbm.at[icv], cbuf, sem)
        cp_w.wait()
        cp_c.wait()
        pltpu.sync_copy(wbuf, out_hbm.at[pl.ds(base, CH), pl.ds(0, DW)])
        pltpu.sync_copy(cbuf, out_hbm.at[pl.ds(base, CH), pl.ds(DW, DC)])


def kernel(seq_inputs, char_seq_inputs, W_word, W_char):
    iw = seq_inputs.reshape(N).astype(jnp.int32)
    ic = char_seq_inputs.reshape(N).astype(jnp.int32)
    mesh = plsc.VectorSubcoreMesh(core_axis_name="c", subcore_axis_name="s")
    run = functools.partial(
        pl.kernel,
        out_type=jax.ShapeDtypeStruct((N, DO), jnp.float32),
        mesh=mesh,
        scratch_types=[
            pltpu.VMEM((CH,), jnp.int32),
            pltpu.VMEM((CH,), jnp.int32),
            pltpu.VMEM((CH, DW), jnp.float32),
            pltpu.VMEM((CH, DC), jnp.float32),
            pltpu.SemaphoreType.DMA,
        ],
    )(_encode)
    out = run(iw, ic, W_word, W_char)
    return out.reshape(B, S, DO)


# speedup vs baseline: 3.1074x; 3.1074x over previous
import functools

import jax
import jax.numpy as jnp
from jax import lax
from jax.experimental import pallas as pl
from jax.experimental.pallas import tpu as pltpu
from jax.experimental.pallas import tpu_sc as plsc

B, S = 4096, 200
N = B * S
DW, DC = 128, 64
DO = DW + DC
NW = 32
TOK_PER_W = N // NW
CH = 128
NCHUNK = TOK_PER_W // CH


def _encode(iw_hbm, ic_hbm, wt_hbm, ct_hbm, out_hbm,
            iwv, icv, wbuf, cbuf, sem):
    wid = lax.axis_index("s") * 2 + lax.axis_index("c")
    w_base = wid * TOK_PER_W

    @pl.loop(0, NCHUNK)
    def _chunk(j):
        base = w_base + j * CH
        pltpu.sync_copy(iw_hbm.at[pl.ds(base, CH)], iwv)
        pltpu.sync_copy(ic_hbm.at[pl.ds(base, CH)], icv)
        cp_w = pltpu.async_copy(wt_hbm.at[iwv], wbuf, sem)
        cp_c = pltpu.async_copy(ct_hbm.at[icv], cbuf, sem)
        cp_w.wait()
        cp_c.wait()
        pltpu.sync_copy(wbuf, out_hbm.at[pl.ds(base, CH), pl.ds(0, DW)])
        pltpu.sync_copy(cbuf, out_hbm.at[pl.ds(base, CH), pl.ds(DW, DC)])


def kernel(seq_inputs, char_seq_inputs, W_word, W_char):
    iw = seq_inputs.reshape(N).astype(jnp.int32)
    ic = char_seq_inputs.reshape(N).astype(jnp.int32)
    mesh = plsc.VectorSubcoreMesh(core_axis_name="c", subcore_axis_name="s")
    run = functools.partial(
        pl.kernel,
        out_type=jax.ShapeDtypeStruct((N, DO), jnp.float32),
        mesh=mesh,
        compiler_params=pltpu.CompilerParams(use_tc_tiling_on_sc=False),
        scratch_types=[
            pltpu.VMEM((CH,), jnp.int32),
            pltpu.VMEM((CH,), jnp.int32),
            pltpu.VMEM((CH, DW), jnp.float32),
            pltpu.VMEM((CH, DC), jnp.float32),
            pltpu.SemaphoreType.DMA,
        ],
    )(_encode)
    out = run(iw, ic, W_word, W_char)
    return out.reshape(B, S, DO)


# speedup vs baseline: 3.5736x; 1.1501x over previous
import functools

import jax
import jax.numpy as jnp
from jax import lax
from jax.experimental import pallas as pl
from jax.experimental.pallas import tpu as pltpu
from jax.experimental.pallas import tpu_sc as plsc

B, S = 4096, 200
N = B * S
DW, DC = 128, 64
DO = DW + DC
NW = 32
TOK_PER_W = N // NW
CH = 128
NCHUNK = TOK_PER_W // CH


def _encode(iw_hbm, ic_hbm, wt_hbm, ct_hbm, out_hbm,
            iwv, icv, wbuf, cbuf, gsem_w, gsem_c, wsem_w, wsem_c):
    wid = lax.axis_index("s") * 2 + lax.axis_index("c")
    w_base = wid * TOK_PER_W

    pltpu.sync_copy(iw_hbm.at[pl.ds(wid * NCHUNK, NCHUNK)], iwv)
    pltpu.sync_copy(ic_hbm.at[pl.ds(wid * NCHUNK, NCHUNK)], icv)

    def gather_start(c, b):
        pltpu.async_copy(wt_hbm.at[iwv.at[c]], wbuf.at[b], gsem_w)
        pltpu.async_copy(ct_hbm.at[icv.at[c]], cbuf.at[b], gsem_c)

    def gather_wait(c, b):
        pltpu.make_async_copy(wt_hbm.at[iwv.at[c]], wbuf.at[b], gsem_w).wait()
        pltpu.make_async_copy(ct_hbm.at[icv.at[c]], cbuf.at[b], gsem_c).wait()

    def write_start(c, b):
        base = w_base + c * CH
        pltpu.async_copy(wbuf.at[b],
                         out_hbm.at[pl.ds(base, CH), pl.ds(0, DW)],
                         wsem_w.at[b])
        pltpu.async_copy(cbuf.at[b],
                         out_hbm.at[pl.ds(base, CH), pl.ds(DW, DC)],
                         wsem_c.at[b])

    def write_wait(b):
        pltpu.make_async_copy(wbuf.at[b],
                              out_hbm.at[pl.ds(0, CH), pl.ds(0, DW)],
                              wsem_w.at[b]).wait()
        pltpu.make_async_copy(cbuf.at[b],
                              out_hbm.at[pl.ds(0, CH), pl.ds(DW, DC)],
                              wsem_c.at[b]).wait()

    gather_start(0, 0)
    gather_wait(0, 0)
    write_start(0, 0)
    gather_start(1, 1)

    @pl.loop(1, NCHUNK - 1, step=2)
    def _steady(c):
        for k in range(2):
            cc = c + k
            b = 1 - k
            bo = 1 - b
            gather_wait(cc, b)
            write_start(cc, b)
            write_wait(bo)
            gather_start(cc + 1, bo)

    gather_wait(NCHUNK - 1, 1)
    write_start(NCHUNK - 1, 1)
    write_wait(0)
    write_wait(1)


def kernel(seq_inputs, char_seq_inputs, W_word, W_char):
    iw = seq_inputs.reshape(N // CH, CH).astype(jnp.int32)
    ic = char_seq_inputs.reshape(N // CH, CH).astype(jnp.int32)
    mesh = plsc.VectorSubcoreMesh(core_axis_name="c", subcore_axis_name="s")
    run = functools.partial(
        pl.kernel,
        out_type=jax.ShapeDtypeStruct((N, DO), jnp.float32),
        mesh=mesh,
        compiler_params=pltpu.CompilerParams(use_tc_tiling_on_sc=False),
        scratch_types=[
            pltpu.VMEM((NCHUNK, CH), jnp.int32),
            pltpu.VMEM((NCHUNK, CH), jnp.int32),
            pltpu.VMEM((2, CH, DW), jnp.float32),
            pltpu.VMEM((2, CH, DC), jnp.float32),
            pltpu.SemaphoreType.DMA,
            pltpu.SemaphoreType.DMA,
            pltpu.SemaphoreType.DMA((2,)),
            pltpu.SemaphoreType.DMA((2,)),
        ],
    )(_encode)
    out = run(iw, ic, W_word, W_char)
    return out.reshape(B, S, DO)


# speedup vs baseline: 3.5987x; 1.0070x over previous
import functools

import jax
import jax.numpy as jnp
from jax import lax
from jax.experimental import pallas as pl
from jax.experimental.pallas import tpu as pltpu
from jax.experimental.pallas import tpu_sc as plsc

B, S = 4096, 200
N = B * S
DW, DC = 128, 64
DO = DW + DC
NW = 32
TOK_PER_W = N // NW
CH = 128
NCHUNK = TOK_PER_W // CH
NBUF = 4


def _encode(iw_hbm, ic_hbm, wt_hbm, ct_hbm, out_hbm,
            iwv, icv, wbuf, cbuf, isem_w, isem_c,
            gsem_w, gsem_c, wsem_w, wsem_c):
    wid = lax.axis_index("s") * 2 + lax.axis_index("c")
    w_base = wid * TOK_PER_W
    row0 = wid * NCHUNK

    def idx_start(c, b):
        pltpu.async_copy(iw_hbm.at[row0 + c], iwv.at[b], isem_w.at[b])
        pltpu.async_copy(ic_hbm.at[row0 + c], icv.at[b], isem_c.at[b])

    def idx_wait(c, b):
        pltpu.make_async_copy(iw_hbm.at[row0 + c], iwv.at[b],
                              isem_w.at[b]).wait()
        pltpu.make_async_copy(ic_hbm.at[row0 + c], icv.at[b],
                              isem_c.at[b]).wait()

    def gather_start(c, b):
        pltpu.async_copy(wt_hbm.at[iwv.at[b]], wbuf.at[b], gsem_w.at[b])
        pltpu.async_copy(ct_hbm.at[icv.at[b]], cbuf.at[b], gsem_c.at[b])

    def gather_wait(c, b):
        pltpu.make_async_copy(wt_hbm.at[iwv.at[b]], wbuf.at[b],
                              gsem_w.at[b]).wait()
        pltpu.make_async_copy(ct_hbm.at[icv.at[b]], cbuf.at[b],
                              gsem_c.at[b]).wait()

    def write_start(c, b):
        base = w_base + c * CH
        pltpu.async_copy(wbuf.at[b],
                         out_hbm.at[pl.ds(base, CH), pl.ds(0, DW)],
                         wsem_w.at[b])
        pltpu.async_copy(cbuf.at[b],
                         out_hbm.at[pl.ds(base, CH), pl.ds(DW, DC)],
                         wsem_c.at[b])

    def write_wait(b):
        pltpu.make_async_copy(wbuf.at[b],
                              out_hbm.at[pl.ds(0, CH), pl.ds(0, DW)],
                              wsem_w.at[b]).wait()
        pltpu.make_async_copy(cbuf.at[b],
                              out_hbm.at[pl.ds(0, CH), pl.ds(DW, DC)],
                              wsem_c.at[b]).wait()

    for c in range(NBUF):
        idx_start(c, c)
    idx_wait(0, 0)
    gather_start(0, 0)
    idx_wait(1, 1)
    gather_start(1, 1)
    gather_wait(0, 0)
    write_start(0, 0)
    idx_wait(2, 2)
    gather_start(2, 2)
    idx_start(4, 0)
    gather_wait(1, 1)
    write_start(1, 1)
    idx_wait(3, 3)
    gather_start(3, 3)

    @pl.loop(2, 2 + 48 * NBUF, step=NBUF)
    def _steady(c):
        for k in range(NBUF):
            cc = c + k
            b = (2 + k) % NBUF
            b2 = (b + 2) % NBUF
            b3 = (b + 3) % NBUF
            gather_wait(cc, b)
            write_start(cc, b)
            write_wait(b2)
            idx_wait(cc + 2, b2)
            gather_start(cc + 2, b2)
            idx_start(cc + 3, b3)

    for cc in range(194, NCHUNK):
        b = cc % NBUF
        b2 = (b + 2) % NBUF
        b3 = (b + 3) % NBUF
        gather_wait(cc, b)
        write_start(cc, b)
        write_wait(b2)
        if cc + 2 < NCHUNK:
            idx_wait(cc + 2, b2)
            gather_start(cc + 2, b2)
        if cc + 3 < NCHUNK:
            idx_start(cc + 3, b3)
    write_wait((NCHUNK - 2) % NBUF)
    write_wait((NCHUNK - 1) % NBUF)


def kernel(seq_inputs, char_seq_inputs, W_word, W_char):
    iw = seq_inputs.reshape(N // CH, CH).astype(jnp.int32)
    ic = char_seq_inputs.reshape(N // CH, CH).astype(jnp.int32)
    mesh = plsc.VectorSubcoreMesh(core_axis_name="c", subcore_axis_name="s")
    run = functools.partial(
        pl.kernel,
        out_type=jax.ShapeDtypeStruct((N, DO), jnp.float32),
        mesh=mesh,
        compiler_params=pltpu.CompilerParams(use_tc_tiling_on_sc=False),
        scratch_types=[
            pltpu.VMEM((NBUF, CH), jnp.int32),
            pltpu.VMEM((NBUF, CH), jnp.int32),
            pltpu.VMEM((NBUF, CH, DW), jnp.float32),
            pltpu.VMEM((NBUF, CH, DC), jnp.float32),
            pltpu.SemaphoreType.DMA((NBUF,)),
            pltpu.SemaphoreType.DMA((NBUF,)),
            pltpu.SemaphoreType.DMA((NBUF,)),
            pltpu.SemaphoreType.DMA((NBUF,)),
            pltpu.SemaphoreType.DMA((NBUF,)),
            pltpu.SemaphoreType.DMA((NBUF,)),
        ],
    )(_encode)
    out = run(iw, ic, W_word, W_char)
    return out.reshape(B, S, DO)


# speedup vs baseline: 5.4038x; 1.5016x over previous
import functools

import jax
import jax.numpy as jnp
from jax import lax
from jax.experimental import pallas as pl
from jax.experimental.pallas import tpu as pltpu
from jax.experimental.pallas import tpu_sc as plsc

B, S = 4096, 200
N = B * S
DW, DC = 128, 64
DO = DW + DC
DP = 2 * DW
NW = 32
TOK_PER_W = N // NW
CH = 128
NCHUNK = TOK_PER_W // CH
NBUF = 2


def _encode(iw_hbm, ic_hbm, wt_hbm, ct_hbm, out_hbm,
            iwv, icv, wbuf, cbuf, isem_w, isem_c,
            gsem_w, gsem_c, wsem_w, wsem_c):
    wid = lax.axis_index("s") * 2 + lax.axis_index("c")
    w_base = wid * TOK_PER_W
    row0 = wid * NCHUNK

    def idx_start(c, b):
        pltpu.async_copy(iw_hbm.at[row0 + c], iwv.at[b], isem_w.at[b])
        pltpu.async_copy(ic_hbm.at[row0 + c], icv.at[b], isem_c.at[b])

    def idx_wait(c, b):
        pltpu.make_async_copy(iw_hbm.at[row0 + c], iwv.at[b],
                              isem_w.at[b]).wait()
        pltpu.make_async_copy(ic_hbm.at[row0 + c], icv.at[b],
                              isem_c.at[b]).wait()

    def gather_start(c, b):
        pltpu.async_copy(wt_hbm.at[iwv.at[b]], wbuf.at[b], gsem_w.at[b])
        pltpu.async_copy(ct_hbm.at[icv.at[b]], cbuf.at[b], gsem_c.at[b])

    def gather_wait(c, b):
        pltpu.make_async_copy(wt_hbm.at[iwv.at[b]], wbuf.at[b],
                              gsem_w.at[b]).wait()
        pltpu.make_async_copy(ct_hbm.at[icv.at[b]], cbuf.at[b],
                              gsem_c.at[b]).wait()

    def write_start(c, b):
        base = w_base + c * CH
        pltpu.async_copy(wbuf.at[b],
                         out_hbm.at[pl.ds(base, CH), pl.ds(0, DW)],
                         wsem_w.at[b])
        pltpu.async_copy(cbuf.at[b],
                         out_hbm.at[pl.ds(base, CH), pl.ds(DW, DW)],
                         wsem_c.at[b])

    def write_wait(b):
        pltpu.make_async_copy(wbuf.at[b],
                              out_hbm.at[pl.ds(0, CH), pl.ds(0, DW)],
                              wsem_w.at[b]).wait()
        pltpu.make_async_copy(cbuf.at[b],
                              out_hbm.at[pl.ds(0, CH), pl.ds(DW, DW)],
                              wsem_c.at[b]).wait()

    idx_start(0, 0)
    idx_start(1, 1)
    idx_wait(0, 0)
    gather_start(0, 0)
    gather_wait(0, 0)
    write_start(0, 0)
    idx_wait(1, 1)
    gather_start(1, 1)
    idx_start(2, 0)

    @pl.loop(1, NCHUNK - 3, step=2)
    def _steady(c):
        for k in range(2):
            cc = c + k
            b = 1 - k
            bo = 1 - b
            gather_wait(cc, b)
            write_start(cc, b)
            write_wait(bo)
            idx_wait(cc + 1, bo)
            gather_start(cc + 1, bo)
            idx_start(cc + 2, b)

    gather_wait(NCHUNK - 3, 1)
    write_start(NCHUNK - 3, 1)
    write_wait(0)
    idx_wait(NCHUNK - 2, 0)
    gather_start(NCHUNK - 2, 0)
    idx_start(NCHUNK - 1, 1)
    gather_wait(NCHUNK - 2, 0)
    write_start(NCHUNK - 2, 0)
    write_wait(1)
    idx_wait(NCHUNK - 1, 1)
    gather_start(NCHUNK - 1, 1)
    gather_wait(NCHUNK - 1, 1)
    write_start(NCHUNK - 1, 1)
    write_wait(0)
    write_wait(1)


def kernel(seq_inputs, char_seq_inputs, W_word, W_char):
    iw = seq_inputs.reshape(N // CH, CH).astype(jnp.int32)
    ic = char_seq_inputs.reshape(N // CH, CH).astype(jnp.int32)
    ct = jnp.pad(W_char, ((0, 0), (0, DW - DC)))
    mesh = plsc.VectorSubcoreMesh(core_axis_name="c", subcore_axis_name="s")
    run = functools.partial(
        pl.kernel,
        out_type=jax.ShapeDtypeStruct((N, DP), jnp.float32),
        mesh=mesh,
        scratch_types=[
            pltpu.VMEM((NBUF, CH), jnp.int32),
            pltpu.VMEM((NBUF, CH), jnp.int32),
            pltpu.VMEM((NBUF, CH, DW), jnp.float32),
            pltpu.VMEM((NBUF, CH, DW), jnp.float32),
            pltpu.SemaphoreType.DMA((NBUF,)),
            pltpu.SemaphoreType.DMA((NBUF,)),
            pltpu.SemaphoreType.DMA((NBUF,)),
            pltpu.SemaphoreType.DMA((NBUF,)),
            pltpu.SemaphoreType.DMA((NBUF,)),
            pltpu.SemaphoreType.DMA((NBUF,)),
        ],
    )(_encode)
    out = run(iw, ic, W_word, ct)
    return out[:, :DO].reshape(B, S, DO)
